# baseline (speedup 1.0000x reference)
import functools

import jax
import jax.numpy as jnp
from jax import lax
from jax.experimental import pallas as pl
from jax.experimental.pallas import tpu as pltpu
from jax.experimental.pallas import tpu_sc as plsc

B = 4096
E = 128
L = 50
PAIR = 2 * L
CP = 104
NC = 2
NS = 16
NW = NC * NS
RPW = B // NW
PPW = RPW // 2
SCH = PPW // 2
LANES = 16


def _reduce_pair(rbuf, h, outb, out_row):
    for s in range(2):
        j0 = h * CP + s * L
        for k in range(E // LANES):
            sl = pl.ds(k * LANES, LANES)
            acc = rbuf[j0, sl]
            for j in range(1, L):
                acc = jnp.maximum(acc, rbuf[j0 + j, sl])
            e = jnp.exp(acc * 2.0)
            outb[out_row + s, sl] = 1.0 - 2.0 / (e + 1.0)


def _make_sc_kernel():
    mesh = plsc.VectorSubcoreMesh(core_axis_name="c", subcore_axis_name="s")

    @functools.partial(
        pl.kernel,
        out_type=jax.ShapeDtypeStruct((B, E), jnp.float32),
        mesh=mesh,
        scratch_types=[
            pltpu.VMEM((PPW * CP,), jnp.int32),
            pltpu.VMEM((2 * CP, E), jnp.float32),
            pltpu.VMEM((2 * CP, E), jnp.float32),
            pltpu.VMEM((RPW, E), jnp.float32),
            pltpu.SemaphoreType.DMA,
            pltpu.SemaphoreType.DMA,
        ],
    )
    def sc_kernel(idx_hbm, table_hbm, out_hbm, slab, rows0, rows1, outb,
                  sem0, sem1):
        wid = lax.axis_index("s") * NC + lax.axis_index("c")
        base = wid * RPW

        slab_off = pl.multiple_of(wid * (PPW * CP), 8)
        pltpu.sync_copy(idx_hbm.at[pl.ds(slab_off, PPW * CP)], slab)

        def start(sc, rbuf, sem):
            for h in range(2):
                pair = 2 * sc + h
                off = pl.multiple_of(pair * CP, 8)
                idxv = slab.at[pl.ds(off, CP)]
                pltpu.async_copy(
                    table_hbm.at[idxv], rbuf.at[pl.ds(h * CP, CP)], sem)

        def drain(rbuf, sem):
            for h in range(2):
                pltpu.make_async_copy(
                    table_hbm.at[pl.ds(0, CP)],
                    rbuf.at[pl.ds(h * CP, CP)], sem).wait()

        start(0, rows0, sem0)
        start(1, rows1, sem1)

        def body(i, carry):
            for t, (rbuf, sem) in enumerate(((rows0, sem0), (rows1, sem1))):
                sc = 2 * i + t
                drain(rbuf, sem)
                for h in range(2):
                    _reduce_pair(rbuf, h, outb, 4 * sc + 2 * h)

                @pl.when(sc + 2 < SCH)
                def _():
                    start(sc + 2, rbuf, sem)
            return carry

        lax.fori_loop(0, SCH // 2, body, 0)

        pltpu.sync_copy(outb, out_hbm.at[pl.ds(base, RPW)])

    return sc_kernel


_sc_kernel = _make_sc_kernel()


@jax.jit
def kernel(input, table):
    inp = input.astype(jnp.int32)
    pairs = inp.reshape(B // 2, PAIR)
    pairs_p = jnp.concatenate([pairs, pairs[:, : CP - PAIR]], axis=1)
    idx_flat = pairs_p.reshape(-1)
    return _sc_kernel(idx_flat, table)


# speedup vs baseline: 1.0031x; 1.0031x over previous
import functools

import jax
import jax.numpy as jnp
from jax import lax
from jax.experimental import pallas as pl
from jax.experimental.pallas import tpu as pltpu
from jax.experimental.pallas import tpu_sc as plsc

B = 4096
E = 128
L = 50
PAIR = 2 * L
CP = 104
NC = 2
NS = 16
NW = NC * NS
RPW = B // NW
PPW = RPW // 2
SCH = PPW // 2
LANES = 16


def _reduce_pair(rbuf, h, outb, out_row):
    for s in range(2):
        j0 = h * CP + s * L
        for k in range(E // LANES):
            sl = pl.ds(k * LANES, LANES)
            acc = rbuf[j0, sl]
            for j in range(1, L):
                acc = jnp.maximum(acc, rbuf[j0 + j, sl])
            e = jnp.exp(acc * 2.0)
            outb[out_row + s, sl] = 1.0 - 2.0 / (e + 1.0)


def _make_sc_kernel():
    mesh = plsc.VectorSubcoreMesh(core_axis_name="c", subcore_axis_name="s")

    @functools.partial(
        pl.kernel,
        out_type=jax.ShapeDtypeStruct((B, E), jnp.float32),
        mesh=mesh,
        scratch_types=[
            pltpu.VMEM((PPW * CP,), jnp.int32),
            pltpu.VMEM((CP, E), jnp.float32),
            pltpu.VMEM((CP, E), jnp.float32),
            pltpu.VMEM((CP, E), jnp.float32),
            pltpu.VMEM((CP, E), jnp.float32),
            pltpu.VMEM((RPW, E), jnp.float32),
            pltpu.SemaphoreType.DMA,
            pltpu.SemaphoreType.DMA,
            pltpu.SemaphoreType.DMA,
            pltpu.SemaphoreType.DMA,
        ],
    )
    def sc_kernel(idx_hbm, table_hbm, out_hbm, slab, rows0, rows1, rows2,
                  rows3, outb, sem0, sem1, sem2, sem3):
        wid = lax.axis_index("s") * NC + lax.axis_index("c")
        base = wid * RPW

        slab_off = pl.multiple_of(wid * (PPW * CP), 8)
        pltpu.sync_copy(idx_hbm.at[pl.ds(slab_off, PPW * CP)], slab)

        bufs = ((rows0, sem0), (rows1, sem1), (rows2, sem2), (rows3, sem3))
        NBUF = len(bufs)

        def start(pair, rbuf, sem):
            off = pl.multiple_of(pair * CP, 8)
            idxv = slab.at[pl.ds(off, CP)]
            pltpu.async_copy(table_hbm.at[idxv], rbuf, sem)

        def drain(rbuf, sem):
            pltpu.make_async_copy(
                table_hbm.at[pl.ds(0, CP)], rbuf, sem).wait()

        for t, (rbuf, sem) in enumerate(bufs):
            start(t, rbuf, sem)

        def body(i, carry):
            for t, (rbuf, sem) in enumerate(bufs):
                pair = NBUF * i + t
                drain(rbuf, sem)
                _reduce_pair(rbuf, 0, outb, 2 * pair)

                @pl.when(pair + NBUF < PPW)
                def _():
                    start(pair + NBUF, rbuf, sem)
            return carry

        lax.fori_loop(0, PPW // NBUF, body, 0)

        pltpu.sync_copy(outb, out_hbm.at[pl.ds(base, RPW)])

    return sc_kernel


_sc_kernel = _make_sc_kernel()


@jax.jit
def kernel(input, table):
    inp = input.astype(jnp.int32)
    pairs = inp.reshape(B // 2, PAIR)
    pairs_p = jnp.concatenate([pairs, pairs[:, : CP - PAIR]], axis=1)
    idx_flat = pairs_p.reshape(-1)
    return _sc_kernel(idx_flat, table)


# speedup vs baseline: 1.0734x; 1.0701x over previous
import functools

import jax
import jax.numpy as jnp
from jax import lax
from jax.experimental import pallas as pl
from jax.experimental.pallas import tpu as pltpu
from jax.experimental.pallas import tpu_sc as plsc

B = 4096
E = 128
L = 50
PAIR = 2 * L
CP = 104
NC = 2
NS = 16
NW = NC * NS
RPW = B // NW
PPW = RPW // 2
SCH = PPW // 2
LANES = 16


def _reduce_pair(rbuf, h, outb, out_row):
    for s in range(2):
        j0 = h * CP + s * L
        for k in range(E // LANES):
            sl = pl.ds(k * LANES, LANES)
            acc = rbuf[j0, sl]
            for j in range(1, L):
                acc = jnp.maximum(acc, rbuf[j0 + j, sl])
            e = jnp.exp(acc * 2.0)
            outb[out_row + s, sl] = 1.0 - 2.0 / (e + 1.0)


def _make_sc_kernel():
    mesh = plsc.VectorSubcoreMesh(core_axis_name="c", subcore_axis_name="s")

    @functools.partial(
        pl.kernel,
        out_type=jax.ShapeDtypeStruct((B, E), jnp.float32),
        mesh=mesh,
        scratch_types=[
            pltpu.VMEM((PPW * CP,), jnp.int32),
            pltpu.VMEM((CP, E), jnp.float32),
            pltpu.VMEM((CP, E), jnp.float32),
            pltpu.VMEM((RPW, E), jnp.float32),
            pltpu.SemaphoreType.DMA,
            pltpu.SemaphoreType.DMA,
        ],
    )
    def sc_kernel(idx_hbm, table_hbm, out_hbm, slab, rows0, rows1,
                  outb, sem0, sem1):
        wid = lax.axis_index("s") * NC + lax.axis_index("c")
        base = wid * RPW

        slab_off = pl.multiple_of(wid * (PPW * CP), 8)
        pltpu.sync_copy(idx_hbm.at[pl.ds(slab_off, PPW * CP)], slab)

        bufs = ((rows0, sem0), (rows1, sem1))
        NBUF = len(bufs)

        def start(pair, rbuf, sem):
            off = pl.multiple_of(pair * CP, 8)
            idxv = slab.at[pl.ds(off, CP)]
            pltpu.async_copy(table_hbm.at[idxv], rbuf, sem)

        def drain(rbuf, sem):
            pltpu.make_async_copy(
                table_hbm.at[pl.ds(0, CP)], rbuf, sem).wait()

        for t, (rbuf, sem) in enumerate(bufs):
            start(t, rbuf, sem)

        def body(i, carry):
            for t, (rbuf, sem) in enumerate(bufs):
                pair = NBUF * i + t
                drain(rbuf, sem)
                _reduce_pair(rbuf, 0, outb, 2 * pair)
                start(pair + NBUF, rbuf, sem)
            return carry

        lax.fori_loop(0, PPW // NBUF - 1, body, 0)

        for t, (rbuf, sem) in enumerate(bufs):
            pair = PPW - NBUF + t
            drain(rbuf, sem)
            _reduce_pair(rbuf, 0, outb, 2 * pair)

        pltpu.sync_copy(outb, out_hbm.at[pl.ds(base, RPW)])

    return sc_kernel


_sc_kernel = _make_sc_kernel()


@jax.jit
def kernel(input, table):
    inp = input.astype(jnp.int32)
    pairs = inp.reshape(B // 2, PAIR)
    pairs_p = jnp.concatenate([pairs, pairs[:, : CP - PAIR]], axis=1)
    idx_flat = pairs_p.reshape(-1)
    return _sc_kernel(idx_flat, table)


# speedup vs baseline: 1.2675x; 1.1808x over previous
import functools

import jax
import jax.numpy as jnp
from jax import lax
from jax.experimental import pallas as pl
from jax.experimental.pallas import tpu as pltpu
from jax.experimental.pallas import tpu_sc as plsc

B = 4096
E = 128
L = 50
LP = 56
NC = 2
NS = 16
NW = NC * NS
RPW = B // NW
LANES = 16


def _reduce_block(rbuf, outb, r):
    for k in range(E // LANES):
        sl = pl.ds(k * LANES, LANES)
        acc0 = rbuf[0, sl]
        acc1 = rbuf[1, sl]
        for j in range(2, LP, 2):
            acc0 = jnp.maximum(acc0, rbuf[j, sl])
            acc1 = jnp.maximum(acc1, rbuf[j + 1, sl])
        acc = jnp.maximum(acc0, acc1)
        e = jnp.exp(acc * 2.0)
        outb[r, sl] = 1.0 - 2.0 / (e + 1.0)


def _make_sc_kernel():
    mesh = plsc.VectorSubcoreMesh(core_axis_name="c", subcore_axis_name="s")

    @functools.partial(
        pl.kernel,
        out_type=jax.ShapeDtypeStruct((B, E), jnp.float32),
        mesh=mesh,
        scratch_types=[
            pltpu.VMEM((RPW * LP,), jnp.int32),
            pltpu.VMEM((LP, E), jnp.float32),
            pltpu.VMEM((LP, E), jnp.float32),
            pltpu.VMEM((RPW, E), jnp.float32),
            pltpu.SemaphoreType.DMA,
            pltpu.SemaphoreType.DMA,
        ],
    )
    def sc_kernel(idx_hbm, table_hbm, out_hbm, slab, rows0, rows1, outb,
                  sem0, sem1):
        wid = lax.axis_index("s") * NC + lax.axis_index("c")
        base = wid * RPW

        slab_off = pl.multiple_of(base * LP, 8)
        pltpu.sync_copy(idx_hbm.at[pl.ds(slab_off, RPW * LP)], slab)

        def start(c, rbuf, sem):
            off = pl.multiple_of(c * LP, 8)
            idxv = slab.at[pl.ds(off, LP)]
            pltpu.async_copy(table_hbm.at[idxv], rbuf, sem)

        def wait(rbuf, sem):
            pltpu.make_async_copy(
                table_hbm.at[pl.ds(0, LP)], rbuf, sem).wait()

        start(0, rows0, sem0)
        start(1, rows1, sem1)

        def body(i, carry):
            a = 2 * i
            wait(rows0, sem0)
            _reduce_block(rows0, outb, a)
            start(a + 2, rows0, sem0)
            wait(rows1, sem1)
            _reduce_block(rows1, outb, a + 1)
            start(a + 3, rows1, sem1)
            return carry

        lax.fori_loop(0, RPW // 2 - 1, body, 0)

        wait(rows0, sem0)
        _reduce_block(rows0, outb, RPW - 2)
        wait(rows1, sem1)
        _reduce_block(rows1, outb, RPW - 1)

        pltpu.sync_copy(outb, out_hbm.at[pl.ds(base, RPW)])

    return sc_kernel


_sc_kernel = _make_sc_kernel()


@jax.jit
def kernel(input, table):
    inp = input.astype(jnp.int32)
    inp_p = jnp.concatenate([inp, inp[:, : LP - L]], axis=1)
    idx_flat = inp_p.reshape(-1)
    return _sc_kernel(idx_flat, table)
